# baseline (speedup 1.0000x reference)
import functools

import jax
import jax.numpy as jnp
from jax import lax
from jax.experimental import pallas as pl
from jax.experimental.pallas import tpu as pltpu
from jax.experimental.pallas import tpu_sc as plsc

N_ENT = 10000
N_REL = 500
D = 128
ROW_BLK = 2000
NPAD = 10240

NT = 16
E_HALF = 160000
EPT = E_HALF // NT
NBATCH = 79
EPT_PAD = NBATCH * 128

NCHUNK = 4
CH_N = NPAD // NCHUNK
CBSZ = CH_N * D
CH_FLAT = 4 * CBSZ
CH_TOT = CH_FLAT + NT * 128
CH_ZERO_PT = CH_TOT // NT
ZU = CH_ZERO_PT // 16
CB_PT = CBSZ // NT
RD_UNIT = CB_PT // 2
SHALF = NPAD * D

DEG_TOT = 12288
DEG_PT = 10240
BIGKEY = 1 << 30


def _sc_scatter_kernel(ei_hbm, et_hbm, zc_hbm,
                       s00, s01, s02, s03, s10, s11, s12, s13,
                       keyb, auxb, zb, zerob, idxb, onesb, tmpb, ztb,
                       s_chunk, deg_hist):
    c = lax.axis_index("c")
    t = lax.axis_index("s")
    ebase = c * E_HALF + t * EPT
    lanes = lax.iota(jnp.int32, 16)

    def zb_body(i, _):
        zerob[pl.ds(i * 16, 16)] = jnp.zeros((16,), jnp.float32)
        return 0
    lax.fori_loop(0, 768 // 16, zb_body, 0)
    for g in range(8):
        onesb[pl.ds(g * 16, 16)] = jnp.ones((16,), jnp.float32)

    pltpu.sync_copy(ei_hbm.at[pl.ds(ebase, EPT)], keyb.at[pl.ds(0, EPT)])
    for p in range(7):
        keyb[pl.ds(EPT + p * 16, 16)] = N_ENT + t * 128 + p * 16 + lanes

    pltpu.sync_copy(zerob.at[pl.ds(0, 768)],
                    deg_hist.at[pl.ds(t * 768, 768)])
    plsc.subcore_barrier()

    def hist_body(b, _):
        for g in range(8):
            idxb[pl.ds(g * 16, 16)] = keyb[pl.ds(b * 128 + g * 16, 16)]
        pltpu.sync_copy(onesb, deg_hist.at[idxb], add=True)
        return 0
    lax.fori_loop(0, NBATCH, hist_body, 0)
    plsc.subcore_barrier()

    pltpu.sync_copy(deg_hist.at[pl.ds(t * 640, 640)], tmpb)

    def dinv_body(i, _):
        x = tmpb[pl.ds(i * 16, 16)]
        y = jnp.full((16,), 1.0, jnp.float32)
        for k in range(1, 11):
            y = jnp.where(x >= float(0.5 * 4 ** k), float(2.0 ** (-k)), y)
        for _ in range(5):
            y = y * (1.5 - 0.5 * x * y * y)
        tmpb[pl.ds(i * 16, 16)] = jnp.where(x >= 1.0, y, 0.0)
        return 0
    lax.fori_loop(0, 640 // 16, dinv_body, 0)
    pltpu.sync_copy(tmpb, deg_hist.at[pl.ds(t * 640, 640)])
    plsc.subcore_barrier()

    for p in range(7):
        auxb[pl.ds(EPT + p * 16, 16)] = jnp.zeros((16,), jnp.int32)
    pltpu.sync_copy(ei_hbm.at[pl.ds(2 * E_HALF + ebase, EPT)],
                    auxb.at[pl.ds(0, EPT)])

    def z_body(b, _):
        for g in range(8):
            idxb[pl.ds(g * 16, 16)] = auxb[pl.ds(b * 128 + g * 16, 16)]
        pltpu.sync_copy(deg_hist.at[idxb], zb.at[pl.ds(b * 128, 128)])
        return 0
    lax.fori_loop(0, NBATCH, z_body, 0)

    def z2_body(b, _):
        for g in range(8):
            idxb[pl.ds(g * 16, 16)] = keyb[pl.ds(b * 128 + g * 16, 16)]
        pltpu.sync_copy(deg_hist.at[idxb], ztb)
        for g in range(8):
            zb[pl.ds(b * 128 + g * 16, 16)] = (
                zb[pl.ds(b * 128 + g * 16, 16)] * ztb[pl.ds(g * 16, 16)])
        return 0
    lax.fori_loop(0, NBATCH, z2_body, 0)

    pltpu.sync_copy(et_hbm.at[pl.ds(ebase, EPT)], auxb.at[pl.ds(0, EPT)])

    def key_body(i, _):
        ty = auxb[pl.ds(i * 16, 16)]
        keyb[pl.ds(i * 16, 16)] = (
            lax.shift_left(keyb[pl.ds(i * 16, 16)], 7) + (ty & 127))
        auxb[pl.ds(i * 16, 16)] = lax.shift_right_logical(ty, 7)
        return 0
    lax.fori_loop(0, EPT // 16, key_body, 0)
    for p in range(7):
        keyb[pl.ds(EPT + p * 16, 16)] = jnp.full((16,), BIGKEY, jnp.int32)
        auxb[pl.ds(EPT + p * 16, 16)] = jnp.zeros((16,), jnp.int32)

    def chunk_body(ci, _):
        cbase = ci * CBSZ

        pltpu.sync_copy(zc_hbm.at[pl.ds(t * CH_ZERO_PT, CH_ZERO_PT)],
                        s_chunk.at[pl.ds(t * CH_ZERO_PT, CH_ZERO_PT)])
        plsc.subcore_barrier()

        def batch_body(b, _):
            for g in range(8):
                k16 = keyb[pl.ds(b * 128 + g * 16, 16)]
                cb16 = auxb[pl.ds(b * 128 + g * 16, 16)]
                local = k16 - cbase
                m = (local >= 0) & (local < CBSZ)
                dmp = CH_FLAT + t * 128 + g * 16 + lanes
                idxb[pl.ds(g * 16, 16)] = jnp.where(m, local + cb16 * CBSZ,
                                                    dmp)
            pltpu.sync_copy(zb.at[pl.ds(b * 128, 128)],
                            s_chunk.at[idxb], add=True)
            return 0
        lax.fori_loop(0, NBATCH, batch_body, 0)
        plsc.subcore_barrier()

        for cb, (sa, sb_) in enumerate(((s00, s10), (s01, s11),
                                        (s02, s12), (s03, s13))):
            srco = cb * CBSZ + t * CB_PT
            dsto = ci * CBSZ + t * CB_PT

            @pl.when(c == 0)
            def _(cb=cb, sa=sa, srco=srco, dsto=dsto):
                pltpu.sync_copy(s_chunk.at[pl.ds(srco, CB_PT)],
                                sa.at[pl.ds(dsto, CB_PT)])

            @pl.when(c == 1)
            def _(cb=cb, sb_=sb_, srco=srco, dsto=dsto):
                pltpu.sync_copy(s_chunk.at[pl.ds(srco, CB_PT)],
                                sb_.at[pl.ds(dsto, CB_PT)])
        plsc.subcore_barrier()
        return 0
    lax.fori_loop(0, NCHUNK, chunk_body, 0)


def _sc_scatter(edge_index, edge_type):
    mesh = plsc.VectorSubcoreMesh(core_axis_name="c", subcore_axis_name="s")
    kern = functools.partial(
        pl.kernel,
        mesh=mesh,
        out_type=[jax.ShapeDtypeStruct((SHALF,), jnp.float32)
                  for _ in range(8)],
        scratch_types=[
            pltpu.VMEM((EPT_PAD,), jnp.int32),
            pltpu.VMEM((EPT_PAD,), jnp.int32),
            pltpu.VMEM((EPT_PAD,), jnp.float32),
            pltpu.VMEM((768,), jnp.float32),
            pltpu.VMEM((128,), jnp.int32),
            pltpu.VMEM((128,), jnp.float32),
            pltpu.VMEM((640,), jnp.float32),
            pltpu.VMEM((128,), jnp.float32),
            pltpu.VMEM_SHARED((CH_TOT,), jnp.float32),
            pltpu.VMEM_SHARED((DEG_TOT,), jnp.float32),
        ],
    )(_sc_scatter_kernel)
    zeros = jnp.zeros((CH_TOT,), jnp.float32)
    return kern(edge_index.reshape(-1), edge_type, zeros)


def _mm_bn_kernel(ri_ref, ro_ref, wi_ref, wo_ref,
                  si0, si1, si2, si3, so0, so1, so2, so3,
                  x_ref, stats_ref, ti_ref, to_ref, acc_ref):
    step = pl.program_id(0)

    @pl.when(step == 0)
    def _():
        ti_ref[...] = jnp.dot(ri_ref[...], wi_ref[...],
                              preferred_element_type=jnp.float32)
        to_ref[...] = jnp.dot(ro_ref[...], wo_ref[...],
                              preferred_element_type=jnp.float32)
        acc_ref[...] = jnp.zeros_like(acc_ref)

    xi = jnp.dot(si0[...], ti_ref[pl.ds(0, 128), :],
                 preferred_element_type=jnp.float32)
    xo = jnp.dot(so0[...], to_ref[pl.ds(0, 128), :],
                 preferred_element_type=jnp.float32)
    for cb, (si, so) in enumerate(((si1, so1), (si2, so2), (si3, so3)),
                                  start=1):
        xi = xi + jnp.dot(si[...], ti_ref[pl.ds(cb * 128, 128), :],
                          preferred_element_type=jnp.float32)
        xo = xo + jnp.dot(so[...], to_ref[pl.ds(cb * 128, 128), :],
                          preferred_element_type=jnp.float32)
    x = 0.5 * (xi + xo)
    x_ref[...] = x
    acc_ref[0, :] += jnp.sum(x, axis=0)
    acc_ref[1, :] += jnp.sum(x * x, axis=0)

    @pl.when(step == pl.num_programs(0) - 1)
    def _():
        stats_ref[...] = acc_ref[...]


def _bn_apply_kernel(x_ref, stats_ref, gamma_ref, beta_ref, out_ref):
    mean = stats_ref[0, :] * (1.0 / N_ENT)
    var = stats_ref[1, :] * (1.0 / N_ENT) - mean * mean
    scale = gamma_ref[...] * jax.lax.rsqrt(var + 1e-5)
    out_ref[...] = jnp.tanh((x_ref[...] - mean) * scale + beta_ref[...])


def _dense_stage(rel_in_p, rel_out_p, w_in, w_out, gamma, beta, s_parts):
    nb = N_ENT // ROW_BLK
    sblk = pl.BlockSpec((ROW_BLK, D), lambda i: (i, 0))
    x, stats = pl.pallas_call(
        _mm_bn_kernel,
        grid=(nb,),
        in_specs=[
            pl.BlockSpec((512, D), lambda i: (0, 0)),
            pl.BlockSpec((512, D), lambda i: (0, 0)),
            pl.BlockSpec((D, D), lambda i: (0, 0)),
            pl.BlockSpec((D, D), lambda i: (0, 0)),
        ] + [sblk] * 8,
        out_specs=[
            pl.BlockSpec((ROW_BLK, D), lambda i: (i, 0)),
            pl.BlockSpec((2, D), lambda i: (0, 0)),
        ],
        out_shape=[
            jax.ShapeDtypeStruct((N_ENT, D), jnp.float32),
            jax.ShapeDtypeStruct((2, D), jnp.float32),
        ],
        scratch_shapes=[
            pltpu.VMEM((512, D), jnp.float32),
            pltpu.VMEM((512, D), jnp.float32),
            pltpu.VMEM((2, D), jnp.float32),
        ],
    )(rel_in_p, rel_out_p, w_in, w_out, *s_parts)

    res = pl.pallas_call(
        _bn_apply_kernel,
        grid=(nb,),
        in_specs=[
            pl.BlockSpec((ROW_BLK, D), lambda i: (i, 0)),
            pl.BlockSpec((2, D), lambda i: (0, 0)),
            pl.BlockSpec((D,), lambda i: (0,)),
            pl.BlockSpec((D,), lambda i: (0,)),
        ],
        out_specs=pl.BlockSpec((ROW_BLK, D), lambda i: (i, 0)),
        out_shape=jax.ShapeDtypeStruct((N_ENT, D), jnp.float32),
    )(x, stats, gamma, beta)
    return res


def kernel(rel_embed, rel_embed_in, rel_embed_out, w_in, w_out, gamma, beta,
           edge_index, edge_type):
    outs = _sc_scatter(edge_index, edge_type)
    s_parts = [o.reshape(NPAD, D) for o in outs]
    pad = ((0, 512 - N_REL), (0, 0))
    rel_in_p = jnp.pad(rel_embed_in, pad)
    rel_out_p = jnp.pad(rel_embed_out, pad)
    res = _dense_stage(rel_in_p, rel_out_p, w_in, w_out, gamma, beta,
                       s_parts)
    return (res, rel_embed)


# speedup vs baseline: 1.0573x; 1.0573x over previous
import functools

import jax
import jax.numpy as jnp
from jax import lax
from jax.experimental import pallas as pl
from jax.experimental.pallas import tpu as pltpu
from jax.experimental.pallas import tpu_sc as plsc

N_ENT = 10000
N_REL = 500
D = 128
ROW_BLK = 2000
NPAD = 10240

NT = 16
E_HALF = 160000
EPT = E_HALF // NT
NBATCH = 79
EPT_PAD = NBATCH * 128

NCHUNK = 4
CH_N = NPAD // NCHUNK
CBSZ = CH_N * D
CH_FLAT = 4 * CBSZ
CH_TOT = CH_FLAT + NT * 128
CH_ZERO_PT = CH_TOT // NT
ZU = CH_ZERO_PT // 16
CB_PT = CBSZ // NT
RD_UNIT = CB_PT // 2
SHALF = NPAD * D

DEG_TOT = 12288
DEG_PT = 10240
BIGKEY = 1 << 30


def _sc_scatter_kernel(ei_hbm, et_hbm, zc_hbm,
                       s00, s01, s02, s03, s10, s11, s12, s13,
                       keyb, auxb, zb, zerob, idxb, onesb, tmpb, ztb,
                       s_chunk, deg_hist):
    c = lax.axis_index("c")
    t = lax.axis_index("s")
    ebase = c * E_HALF + t * EPT
    lanes = lax.iota(jnp.int32, 16)

    def zb_body(i, _):
        zerob[pl.ds(i * 16, 16)] = jnp.zeros((16,), jnp.float32)
        return 0
    lax.fori_loop(0, ZU // 16, zb_body, 0)
    for g in range(8):
        onesb[pl.ds(g * 16, 16)] = jnp.ones((16,), jnp.float32)

    pltpu.sync_copy(ei_hbm.at[pl.ds(ebase, EPT)], keyb.at[pl.ds(0, EPT)])
    for p in range(7):
        keyb[pl.ds(EPT + p * 16, 16)] = N_ENT + t * 128 + p * 16 + lanes

    pltpu.sync_copy(zerob.at[pl.ds(0, 768)],
                    deg_hist.at[pl.ds(t * 768, 768)])
    plsc.subcore_barrier()

    def hist_body(b, _):
        for g in range(8):
            idxb[pl.ds(g * 16, 16)] = keyb[pl.ds(b * 128 + g * 16, 16)]
        pltpu.sync_copy(onesb, deg_hist.at[idxb], add=True)
        return 0
    lax.fori_loop(0, NBATCH, hist_body, 0)
    plsc.subcore_barrier()

    pltpu.sync_copy(deg_hist.at[pl.ds(t * 640, 640)], tmpb)

    def dinv_body(i, _):
        x = tmpb[pl.ds(i * 16, 16)]
        y = jnp.full((16,), 1.0, jnp.float32)
        for k in range(1, 11):
            y = jnp.where(x >= float(0.5 * 4 ** k), float(2.0 ** (-k)), y)
        for _ in range(5):
            y = y * (1.5 - 0.5 * x * y * y)
        tmpb[pl.ds(i * 16, 16)] = jnp.where(x >= 1.0, y, 0.0)
        return 0
    lax.fori_loop(0, 640 // 16, dinv_body, 0)
    pltpu.sync_copy(tmpb, deg_hist.at[pl.ds(t * 640, 640)])
    plsc.subcore_barrier()

    for p in range(7):
        auxb[pl.ds(EPT + p * 16, 16)] = jnp.zeros((16,), jnp.int32)
    pltpu.sync_copy(ei_hbm.at[pl.ds(2 * E_HALF + ebase, EPT)],
                    auxb.at[pl.ds(0, EPT)])

    def z_body(b, _):
        for g in range(8):
            idxb[pl.ds(g * 16, 16)] = auxb[pl.ds(b * 128 + g * 16, 16)]
        pltpu.sync_copy(deg_hist.at[idxb], zb.at[pl.ds(b * 128, 128)])
        return 0
    lax.fori_loop(0, NBATCH, z_body, 0)

    def z2_body(b, _):
        for g in range(8):
            idxb[pl.ds(g * 16, 16)] = keyb[pl.ds(b * 128 + g * 16, 16)]
        pltpu.sync_copy(deg_hist.at[idxb], ztb)
        for g in range(8):
            zb[pl.ds(b * 128 + g * 16, 16)] = (
                zb[pl.ds(b * 128 + g * 16, 16)] * ztb[pl.ds(g * 16, 16)])
        return 0
    lax.fori_loop(0, NBATCH, z2_body, 0)

    pltpu.sync_copy(et_hbm.at[pl.ds(ebase, EPT)], auxb.at[pl.ds(0, EPT)])

    def key_body(i, _):
        ty = auxb[pl.ds(i * 16, 16)]
        keyb[pl.ds(i * 16, 16)] = (
            lax.shift_left(keyb[pl.ds(i * 16, 16)], 7) + (ty & 127))
        auxb[pl.ds(i * 16, 16)] = lax.shift_right_logical(ty, 7)
        return 0
    lax.fori_loop(0, EPT // 16, key_body, 0)
    for p in range(7):
        keyb[pl.ds(EPT + p * 16, 16)] = jnp.full((16,), BIGKEY, jnp.int32)
        auxb[pl.ds(EPT + p * 16, 16)] = jnp.zeros((16,), jnp.int32)

    def chunk_body(ci, _):
        cbase = ci * CBSZ

        def zero_body(j, _):
            pltpu.sync_copy(zerob.at[pl.ds(0, ZU)],
                            s_chunk.at[pl.ds(t * CH_ZERO_PT + j * ZU, ZU)])
            return 0
        lax.fori_loop(0, CH_ZERO_PT // ZU, zero_body, 0)
        plsc.subcore_barrier()

        def batch_body(b, _):
            for g in range(8):
                k16 = keyb[pl.ds(b * 128 + g * 16, 16)]
                cb16 = auxb[pl.ds(b * 128 + g * 16, 16)]
                local = k16 - cbase
                m = (local >= 0) & (local < CBSZ)
                dmp = CH_FLAT + t * 128 + g * 16 + lanes
                idxb[pl.ds(g * 16, 16)] = jnp.where(m, local + cb16 * CBSZ,
                                                    dmp)
            pltpu.sync_copy(zb.at[pl.ds(b * 128, 128)],
                            s_chunk.at[idxb], add=True)
            return 0
        lax.fori_loop(0, NBATCH, batch_body, 0)
        plsc.subcore_barrier()

        for cb, (sa, sb_) in enumerate(((s00, s10), (s01, s11),
                                        (s02, s12), (s03, s13))):
            srco = cb * CBSZ + t * CB_PT
            dsto = ci * CBSZ + t * CB_PT

            @pl.when(c == 0)
            def _(cb=cb, sa=sa, srco=srco, dsto=dsto):
                pltpu.sync_copy(s_chunk.at[pl.ds(srco, CB_PT)],
                                sa.at[pl.ds(dsto, CB_PT)])

            @pl.when(c == 1)
            def _(cb=cb, sb_=sb_, srco=srco, dsto=dsto):
                pltpu.sync_copy(s_chunk.at[pl.ds(srco, CB_PT)],
                                sb_.at[pl.ds(dsto, CB_PT)])
        plsc.subcore_barrier()
        return 0
    lax.fori_loop(0, NCHUNK, chunk_body, 0)


def _sc_scatter(edge_index, edge_type):
    mesh = plsc.VectorSubcoreMesh(core_axis_name="c", subcore_axis_name="s")
    kern = functools.partial(
        pl.kernel,
        mesh=mesh,
        out_type=[jax.ShapeDtypeStruct((SHALF,), jnp.float32)
                  for _ in range(8)],
        scratch_types=[
            pltpu.VMEM((EPT_PAD,), jnp.int32),
            pltpu.VMEM((EPT_PAD,), jnp.int32),
            pltpu.VMEM((EPT_PAD,), jnp.float32),
            pltpu.VMEM((ZU,), jnp.float32),
            pltpu.VMEM((128,), jnp.int32),
            pltpu.VMEM((128,), jnp.float32),
            pltpu.VMEM((640,), jnp.float32),
            pltpu.VMEM((128,), jnp.float32),
            pltpu.VMEM_SHARED((CH_TOT,), jnp.float32),
            pltpu.VMEM_SHARED((DEG_TOT,), jnp.float32),
        ],
    )(_sc_scatter_kernel)
    zeros = jnp.zeros((CH_TOT,), jnp.float32)
    return kern(edge_index.reshape(-1), edge_type, zeros)


def _mm_bn_kernel(ri_ref, ro_ref, wi_ref, wo_ref,
                  si0, si1, si2, si3, so0, so1, so2, so3,
                  x_ref, stats_ref, ti_ref, to_ref, acc_ref):
    step = pl.program_id(0)

    @pl.when(step == 0)
    def _():
        ti_ref[...] = jnp.dot(ri_ref[...], wi_ref[...],
                              preferred_element_type=jnp.float32)
        to_ref[...] = jnp.dot(ro_ref[...], wo_ref[...],
                              preferred_element_type=jnp.float32)
        acc_ref[...] = jnp.zeros_like(acc_ref)

    xi = jnp.dot(si0[...], ti_ref[pl.ds(0, 128), :],
                 preferred_element_type=jnp.float32)
    xo = jnp.dot(so0[...], to_ref[pl.ds(0, 128), :],
                 preferred_element_type=jnp.float32)
    for cb, (si, so) in enumerate(((si1, so1), (si2, so2), (si3, so3)),
                                  start=1):
        xi = xi + jnp.dot(si[...], ti_ref[pl.ds(cb * 128, 128), :],
                          preferred_element_type=jnp.float32)
        xo = xo + jnp.dot(so[...], to_ref[pl.ds(cb * 128, 128), :],
                          preferred_element_type=jnp.float32)
    x = 0.5 * (xi + xo)
    x_ref[...] = x
    acc_ref[0, :] += jnp.sum(x, axis=0)
    acc_ref[1, :] += jnp.sum(x * x, axis=0)

    @pl.when(step == pl.num_programs(0) - 1)
    def _():
        stats_ref[...] = acc_ref[...]


def _bn_apply_kernel(x_ref, stats_ref, gamma_ref, beta_ref, out_ref):
    mean = stats_ref[0, :] * (1.0 / N_ENT)
    var = stats_ref[1, :] * (1.0 / N_ENT) - mean * mean
    scale = gamma_ref[...] * jax.lax.rsqrt(var + 1e-5)
    out_ref[...] = jnp.tanh((x_ref[...] - mean) * scale + beta_ref[...])


def _dense_stage(rel_in_p, rel_out_p, w_in, w_out, gamma, beta, s_parts):
    nb = N_ENT // ROW_BLK
    sblk = pl.BlockSpec((ROW_BLK, D), lambda i: (i, 0))
    x, stats = pl.pallas_call(
        _mm_bn_kernel,
        grid=(nb,),
        in_specs=[
            pl.BlockSpec((512, D), lambda i: (0, 0)),
            pl.BlockSpec((512, D), lambda i: (0, 0)),
            pl.BlockSpec((D, D), lambda i: (0, 0)),
            pl.BlockSpec((D, D), lambda i: (0, 0)),
        ] + [sblk] * 8,
        out_specs=[
            pl.BlockSpec((ROW_BLK, D), lambda i: (i, 0)),
            pl.BlockSpec((2, D), lambda i: (0, 0)),
        ],
        out_shape=[
            jax.ShapeDtypeStruct((N_ENT, D), jnp.float32),
            jax.ShapeDtypeStruct((2, D), jnp.float32),
        ],
        scratch_shapes=[
            pltpu.VMEM((512, D), jnp.float32),
            pltpu.VMEM((512, D), jnp.float32),
            pltpu.VMEM((2, D), jnp.float32),
        ],
    )(rel_in_p, rel_out_p, w_in, w_out, *s_parts)

    res = pl.pallas_call(
        _bn_apply_kernel,
        grid=(nb,),
        in_specs=[
            pl.BlockSpec((ROW_BLK, D), lambda i: (i, 0)),
            pl.BlockSpec((2, D), lambda i: (0, 0)),
            pl.BlockSpec((D,), lambda i: (0,)),
            pl.BlockSpec((D,), lambda i: (0,)),
        ],
        out_specs=pl.BlockSpec((ROW_BLK, D), lambda i: (i, 0)),
        out_shape=jax.ShapeDtypeStruct((N_ENT, D), jnp.float32),
    )(x, stats, gamma, beta)
    return res


def kernel(rel_embed, rel_embed_in, rel_embed_out, w_in, w_out, gamma, beta,
           edge_index, edge_type):
    outs = _sc_scatter(edge_index, edge_type)
    s_parts = [o.reshape(NPAD, D) for o in outs]
    pad = ((0, 512 - N_REL), (0, 0))
    rel_in_p = jnp.pad(rel_embed_in, pad)
    rel_out_p = jnp.pad(rel_embed_out, pad)
    res = _dense_stage(rel_in_p, rel_out_p, w_in, w_out, gamma, beta,
                       s_parts)
    return (res, rel_embed)


# speedup vs baseline: 1.0578x; 1.0005x over previous
import functools

import jax
import jax.numpy as jnp
from jax import lax
from jax.experimental import pallas as pl
from jax.experimental.pallas import tpu as pltpu
from jax.experimental.pallas import tpu_sc as plsc

N_ENT = 10000
N_REL = 500
D = 128
ROW_BLK = 2000
NPAD = 10240

NT = 16
E_HALF = 160000
EPT = E_HALF // NT
NBATCH = 79
EPT_PAD = NBATCH * 128

NCHUNK = 4
CH_N = NPAD // NCHUNK
CBSZ = CH_N * D
CH_FLAT = 4 * CBSZ
CH_TOT = CH_FLAT + NT * 128
CH_ZERO_PT = CH_FLAT // NT
ZU = CH_ZERO_PT // 16
CB_PT = CBSZ // NT
RD_UNIT = CB_PT // 2
SHALF = NPAD * D

DEG_TOT = 12288
DEG_PT = 10240
BIGKEY = 1 << 30


def _sc_scatter_kernel(ei_hbm, et_hbm, zc_hbm,
                       s00, s01, s02, s03, s10, s11, s12, s13,
                       keyb, auxb, zb, zerob, idxb, onesb, tmpb, ztb,
                       s_chunk, deg_hist):
    c = lax.axis_index("c")
    t = lax.axis_index("s")
    ebase = c * E_HALF + t * EPT
    lanes = lax.iota(jnp.int32, 16)

    def zb_body(i, _):
        zerob[pl.ds(i * 16, 16)] = jnp.zeros((16,), jnp.float32)
        return 0
    lax.fori_loop(0, ZU // 16, zb_body, 0)
    for g in range(8):
        onesb[pl.ds(g * 16, 16)] = jnp.ones((16,), jnp.float32)

    pltpu.sync_copy(ei_hbm.at[pl.ds(ebase, EPT)], keyb.at[pl.ds(0, EPT)])
    for p in range(7):
        keyb[pl.ds(EPT + p * 16, 16)] = N_ENT + t * 128 + p * 16 + lanes

    pltpu.sync_copy(zerob.at[pl.ds(0, 768)],
                    deg_hist.at[pl.ds(t * 768, 768)])
    plsc.subcore_barrier()

    def hist_body(b, _):
        for g in range(8):
            idxb[pl.ds(g * 16, 16)] = keyb[pl.ds(b * 128 + g * 16, 16)]
        pltpu.sync_copy(onesb, deg_hist.at[idxb], add=True)
        return 0
    lax.fori_loop(0, NBATCH, hist_body, 0)
    plsc.subcore_barrier()

    pltpu.sync_copy(deg_hist.at[pl.ds(t * 640, 640)], tmpb)

    def dinv_body(i, _):
        x = tmpb[pl.ds(i * 16, 16)]
        y = jnp.full((16,), 1.0, jnp.float32)
        for k in range(1, 11):
            y = jnp.where(x >= float(0.5 * 4 ** k), float(2.0 ** (-k)), y)
        for _ in range(5):
            y = y * (1.5 - 0.5 * x * y * y)
        tmpb[pl.ds(i * 16, 16)] = jnp.where(x >= 1.0, y, 0.0)
        return 0
    lax.fori_loop(0, 640 // 16, dinv_body, 0)
    pltpu.sync_copy(tmpb, deg_hist.at[pl.ds(t * 640, 640)])
    plsc.subcore_barrier()

    for p in range(7):
        auxb[pl.ds(EPT + p * 16, 16)] = jnp.zeros((16,), jnp.int32)
    pltpu.sync_copy(ei_hbm.at[pl.ds(2 * E_HALF + ebase, EPT)],
                    auxb.at[pl.ds(0, EPT)])

    def z_body(b, _):
        for g in range(8):
            idxb[pl.ds(g * 16, 16)] = auxb[pl.ds(b * 128 + g * 16, 16)]
        pltpu.sync_copy(deg_hist.at[idxb], zb.at[pl.ds(b * 128, 128)])
        return 0
    lax.fori_loop(0, NBATCH, z_body, 0)

    def z2_body(b, _):
        for g in range(8):
            idxb[pl.ds(g * 16, 16)] = keyb[pl.ds(b * 128 + g * 16, 16)]
        pltpu.sync_copy(deg_hist.at[idxb], ztb)
        for g in range(8):
            zb[pl.ds(b * 128 + g * 16, 16)] = (
                zb[pl.ds(b * 128 + g * 16, 16)] * ztb[pl.ds(g * 16, 16)])
        return 0
    lax.fori_loop(0, NBATCH, z2_body, 0)

    pltpu.sync_copy(et_hbm.at[pl.ds(ebase, EPT)], auxb.at[pl.ds(0, EPT)])

    def key_body(i, _):
        ty = auxb[pl.ds(i * 16, 16)]
        keyb[pl.ds(i * 16, 16)] = (
            lax.shift_left(keyb[pl.ds(i * 16, 16)], 7) + (ty & 127))
        auxb[pl.ds(i * 16, 16)] = lax.shift_right_logical(ty, 7)
        return 0
    lax.fori_loop(0, EPT // 16, key_body, 0)
    for p in range(7):
        keyb[pl.ds(EPT + p * 16, 16)] = jnp.full((16,), BIGKEY, jnp.int32)
        auxb[pl.ds(EPT + p * 16, 16)] = jnp.zeros((16,), jnp.int32)

    def chunk_body(ci, _):
        cbase = ci * CBSZ

        def zero_body(j, _):
            pltpu.sync_copy(zerob.at[pl.ds(0, ZU)],
                            s_chunk.at[pl.ds(t * CH_ZERO_PT + j * ZU, ZU)])
            return 0
        lax.fori_loop(0, CH_ZERO_PT // ZU, zero_body, 0)
        plsc.subcore_barrier()

        def batch_body(b, _):
            for g in range(8):
                k16 = keyb[pl.ds(b * 128 + g * 16, 16)]
                cb16 = auxb[pl.ds(b * 128 + g * 16, 16)]
                local = k16 - cbase
                m = (local >= 0) & (local < CBSZ)
                dmp = CH_FLAT + t * 128 + g * 16 + lanes
                idxb[pl.ds(g * 16, 16)] = jnp.where(m, local + cb16 * CBSZ,
                                                    dmp)
            pltpu.sync_copy(zb.at[pl.ds(b * 128, 128)],
                            s_chunk.at[idxb], add=True)
            return 0
        lax.fori_loop(0, NBATCH, batch_body, 0)
        plsc.subcore_barrier()

        for cb, (sa, sb_) in enumerate(((s00, s10), (s01, s11),
                                        (s02, s12), (s03, s13))):
            srco = cb * CBSZ + t * CB_PT
            dsto = ci * CBSZ + t * CB_PT

            @pl.when(c == 0)
            def _(cb=cb, sa=sa, srco=srco, dsto=dsto):
                pltpu.sync_copy(s_chunk.at[pl.ds(srco, CB_PT)],
                                sa.at[pl.ds(dsto, CB_PT)])

            @pl.when(c == 1)
            def _(cb=cb, sb_=sb_, srco=srco, dsto=dsto):
                pltpu.sync_copy(s_chunk.at[pl.ds(srco, CB_PT)],
                                sb_.at[pl.ds(dsto, CB_PT)])
        plsc.subcore_barrier()
        return 0
    lax.fori_loop(0, NCHUNK, chunk_body, 0)


def _sc_scatter(edge_index, edge_type):
    mesh = plsc.VectorSubcoreMesh(core_axis_name="c", subcore_axis_name="s")
    kern = functools.partial(
        pl.kernel,
        mesh=mesh,
        out_type=[jax.ShapeDtypeStruct((SHALF,), jnp.float32)
                  for _ in range(8)],
        scratch_types=[
            pltpu.VMEM((EPT_PAD,), jnp.int32),
            pltpu.VMEM((EPT_PAD,), jnp.int32),
            pltpu.VMEM((EPT_PAD,), jnp.float32),
            pltpu.VMEM((ZU,), jnp.float32),
            pltpu.VMEM((128,), jnp.int32),
            pltpu.VMEM((128,), jnp.float32),
            pltpu.VMEM((640,), jnp.float32),
            pltpu.VMEM((128,), jnp.float32),
            pltpu.VMEM_SHARED((CH_TOT,), jnp.float32),
            pltpu.VMEM_SHARED((DEG_TOT,), jnp.float32),
        ],
    )(_sc_scatter_kernel)
    zeros = jnp.zeros((CH_TOT,), jnp.float32)
    return kern(edge_index.reshape(-1), edge_type, zeros)


def _mm_bn_kernel(ri_ref, ro_ref, wi_ref, wo_ref,
                  si0, si1, si2, si3, so0, so1, so2, so3,
                  x_ref, stats_ref, ti_ref, to_ref, acc_ref):
    step = pl.program_id(0)

    @pl.when(step == 0)
    def _():
        ti_ref[...] = jnp.dot(ri_ref[...], wi_ref[...],
                              preferred_element_type=jnp.float32)
        to_ref[...] = jnp.dot(ro_ref[...], wo_ref[...],
                              preferred_element_type=jnp.float32)
        acc_ref[...] = jnp.zeros_like(acc_ref)

    xi = jnp.dot(si0[...], ti_ref[pl.ds(0, 128), :],
                 preferred_element_type=jnp.float32)
    xo = jnp.dot(so0[...], to_ref[pl.ds(0, 128), :],
                 preferred_element_type=jnp.float32)
    for cb, (si, so) in enumerate(((si1, so1), (si2, so2), (si3, so3)),
                                  start=1):
        xi = xi + jnp.dot(si[...], ti_ref[pl.ds(cb * 128, 128), :],
                          preferred_element_type=jnp.float32)
        xo = xo + jnp.dot(so[...], to_ref[pl.ds(cb * 128, 128), :],
                          preferred_element_type=jnp.float32)
    x = 0.5 * (xi + xo)
    x_ref[...] = x
    acc_ref[0, :] += jnp.sum(x, axis=0)
    acc_ref[1, :] += jnp.sum(x * x, axis=0)

    @pl.when(step == pl.num_programs(0) - 1)
    def _():
        stats_ref[...] = acc_ref[...]


def _bn_apply_kernel(x_ref, stats_ref, gamma_ref, beta_ref, out_ref):
    mean = stats_ref[0, :] * (1.0 / N_ENT)
    var = stats_ref[1, :] * (1.0 / N_ENT) - mean * mean
    scale = gamma_ref[...] * jax.lax.rsqrt(var + 1e-5)
    out_ref[...] = jnp.tanh((x_ref[...] - mean) * scale + beta_ref[...])


def _dense_stage(rel_in_p, rel_out_p, w_in, w_out, gamma, beta, s_parts):
    nb = N_ENT // ROW_BLK
    sblk = pl.BlockSpec((ROW_BLK, D), lambda i: (i, 0))
    x, stats = pl.pallas_call(
        _mm_bn_kernel,
        grid=(nb,),
        in_specs=[
            pl.BlockSpec((512, D), lambda i: (0, 0)),
            pl.BlockSpec((512, D), lambda i: (0, 0)),
            pl.BlockSpec((D, D), lambda i: (0, 0)),
            pl.BlockSpec((D, D), lambda i: (0, 0)),
        ] + [sblk] * 8,
        out_specs=[
            pl.BlockSpec((ROW_BLK, D), lambda i: (i, 0)),
            pl.BlockSpec((2, D), lambda i: (0, 0)),
        ],
        out_shape=[
            jax.ShapeDtypeStruct((N_ENT, D), jnp.float32),
            jax.ShapeDtypeStruct((2, D), jnp.float32),
        ],
        scratch_shapes=[
            pltpu.VMEM((512, D), jnp.float32),
            pltpu.VMEM((512, D), jnp.float32),
            pltpu.VMEM((2, D), jnp.float32),
        ],
    )(rel_in_p, rel_out_p, w_in, w_out, *s_parts)

    res = pl.pallas_call(
        _bn_apply_kernel,
        grid=(nb,),
        in_specs=[
            pl.BlockSpec((ROW_BLK, D), lambda i: (i, 0)),
            pl.BlockSpec((2, D), lambda i: (0, 0)),
            pl.BlockSpec((D,), lambda i: (0,)),
            pl.BlockSpec((D,), lambda i: (0,)),
        ],
        out_specs=pl.BlockSpec((ROW_BLK, D), lambda i: (i, 0)),
        out_shape=jax.ShapeDtypeStruct((N_ENT, D), jnp.float32),
    )(x, stats, gamma, beta)
    return res


def kernel(rel_embed, rel_embed_in, rel_embed_out, w_in, w_out, gamma, beta,
           edge_index, edge_type):
    outs = _sc_scatter(edge_index, edge_type)
    s_parts = [o.reshape(NPAD, D) for o in outs]
    pad = ((0, 512 - N_REL), (0, 0))
    rel_in_p = jnp.pad(rel_embed_in, pad)
    rel_out_p = jnp.pad(rel_embed_out, pad)
    res = _dense_stage(rel_in_p, rel_out_p, w_in, w_out, gamma, beta,
                       s_parts)
    return (res, rel_embed)


# speedup vs baseline: 1.3830x; 1.3074x over previous
import functools

import jax
import jax.numpy as jnp
from jax import lax
from jax.experimental import pallas as pl
from jax.experimental.pallas import tpu as pltpu
from jax.experimental.pallas import tpu_sc as plsc

N_ENT = 10000
N_REL = 500
D = 128
ROW_BLK = 2000
NPAD = 10240

NT = 16
E_HALF = 160000
EPT = E_HALF // NT
NBATCH = 79
EPT_PAD = NBATCH * 128

NCHUNK = 4
CH_N = NPAD // NCHUNK
CBSZ = CH_N * D
CH_FLAT = 4 * CBSZ
CH_TOT = CH_FLAT + NT * 128
CH_ZERO_PT = CH_FLAT // NT
ZU = CH_ZERO_PT // 16
CB_PT = CBSZ // NT
RD_UNIT = CB_PT // 2
SHALF = NPAD * D

DEG_TOT = 12288
DEG_PT = 10240
BIGKEY = 1 << 30


def _sc_scatter_kernel(ei_hbm, et_hbm, zc_hbm,
                       s00, s01, s02, s03, s10, s11, s12, s13,
                       keyb, auxb, zb, zerob, idxb0, idxb1, idxb2, idxb3,
                       onesb, tmpb, ztb, s_chunk, deg_hist, dsem):
    c = lax.axis_index("c")
    t = lax.axis_index("s")
    ebase = c * E_HALF + t * EPT
    lanes = lax.iota(jnp.int32, 16)

    def zb_body(i, _):
        zerob[pl.ds(i * 16, 16)] = jnp.zeros((16,), jnp.float32)
        return 0
    lax.fori_loop(0, ZU // 16, zb_body, 0)
    for g in range(8):
        onesb[pl.ds(g * 16, 16)] = jnp.ones((16,), jnp.float32)

    pltpu.sync_copy(ei_hbm.at[pl.ds(ebase, EPT)], keyb.at[pl.ds(0, EPT)])
    for p in range(7):
        keyb[pl.ds(EPT + p * 16, 16)] = N_ENT + t * 128 + p * 16 + lanes

    pltpu.sync_copy(zerob.at[pl.ds(0, 768)],
                    deg_hist.at[pl.ds(t * 768, 768)])
    plsc.subcore_barrier()

    ibufs = (idxb0, idxb1, idxb2, idxb3)

    def hist_quad(q, _):
        descs = []
        for u in range(4):
            b = q * 4 + u
            ib = ibufs[u]
            for g in range(8):
                ib[pl.ds(g * 16, 16)] = keyb[pl.ds(b * 128 + g * 16, 16)]
            descs.append(pltpu.async_copy(onesb, deg_hist.at[ib], dsem,
                                          add=True))
        for dsc in descs:
            dsc.wait()
        return 0
    lax.fori_loop(0, NBATCH // 4, hist_quad, 0)
    for b in range(NBATCH - NBATCH % 4, NBATCH):
        for g in range(8):
            idxb0[pl.ds(g * 16, 16)] = keyb[pl.ds(b * 128 + g * 16, 16)]
        pltpu.sync_copy(onesb, deg_hist.at[idxb0], add=True)
    plsc.subcore_barrier()

    pltpu.sync_copy(deg_hist.at[pl.ds(t * 640, 640)], tmpb)

    def dinv_body(i, _):
        x = tmpb[pl.ds(i * 16, 16)]
        y = jnp.full((16,), 1.0, jnp.float32)
        for k in range(1, 11):
            y = jnp.where(x >= float(0.5 * 4 ** k), float(2.0 ** (-k)), y)
        for _ in range(5):
            y = y * (1.5 - 0.5 * x * y * y)
        tmpb[pl.ds(i * 16, 16)] = jnp.where(x >= 1.0, y, 0.0)
        return 0
    lax.fori_loop(0, 640 // 16, dinv_body, 0)
    pltpu.sync_copy(tmpb, deg_hist.at[pl.ds(t * 640, 640)])
    plsc.subcore_barrier()

    for p in range(7):
        auxb[pl.ds(EPT + p * 16, 16)] = jnp.zeros((16,), jnp.int32)
    pltpu.sync_copy(ei_hbm.at[pl.ds(2 * E_HALF + ebase, EPT)],
                    auxb.at[pl.ds(0, EPT)])

    def z_pair(q, _):
        descs = []
        for u in range(2):
            b = q * 2 + u
            ia, ik = ibufs[2 * u], ibufs[2 * u + 1]
            for g in range(8):
                ia[pl.ds(g * 16, 16)] = auxb[pl.ds(b * 128 + g * 16, 16)]
                ik[pl.ds(g * 16, 16)] = keyb[pl.ds(b * 128 + g * 16, 16)]
            descs.append(pltpu.async_copy(deg_hist.at[ia],
                                          zb.at[pl.ds(b * 128, 128)], dsem))
            descs.append(pltpu.async_copy(deg_hist.at[ik],
                                          ztb.at[pl.ds(u * 128, 128)], dsem))
        for dsc in descs:
            dsc.wait()
        for u in range(2):
            b = q * 2 + u
            for g in range(8):
                zb[pl.ds(b * 128 + g * 16, 16)] = (
                    zb[pl.ds(b * 128 + g * 16, 16)]
                    * ztb[pl.ds(u * 128 + g * 16, 16)])
        return 0
    lax.fori_loop(0, NBATCH // 2, z_pair, 0)
    for b in range(NBATCH - NBATCH % 2, NBATCH):
        for g in range(8):
            idxb0[pl.ds(g * 16, 16)] = auxb[pl.ds(b * 128 + g * 16, 16)]
            idxb1[pl.ds(g * 16, 16)] = keyb[pl.ds(b * 128 + g * 16, 16)]
        pltpu.sync_copy(deg_hist.at[idxb0], zb.at[pl.ds(b * 128, 128)])
        pltpu.sync_copy(deg_hist.at[idxb1], ztb.at[pl.ds(0, 128)])
        for g in range(8):
            zb[pl.ds(b * 128 + g * 16, 16)] = (
                zb[pl.ds(b * 128 + g * 16, 16)] * ztb[pl.ds(g * 16, 16)])

    pltpu.sync_copy(et_hbm.at[pl.ds(ebase, EPT)], auxb.at[pl.ds(0, EPT)])

    def key_body(i, _):
        ty = auxb[pl.ds(i * 16, 16)]
        keyb[pl.ds(i * 16, 16)] = (
            lax.shift_left(keyb[pl.ds(i * 16, 16)], 7) + (ty & 127))
        auxb[pl.ds(i * 16, 16)] = lax.shift_right_logical(ty, 7)
        return 0
    lax.fori_loop(0, EPT // 16, key_body, 0)
    for p in range(7):
        keyb[pl.ds(EPT + p * 16, 16)] = jnp.full((16,), BIGKEY, jnp.int32)
        auxb[pl.ds(EPT + p * 16, 16)] = jnp.zeros((16,), jnp.int32)

    def chunk_body(ci, _):
        cbase = ci * CBSZ

        zdescs = [pltpu.async_copy(
            zerob.at[pl.ds(0, ZU)],
            s_chunk.at[pl.ds(t * CH_ZERO_PT + j * ZU, ZU)], dsem)
            for j in range(CH_ZERO_PT // ZU)]
        for dsc in zdescs:
            dsc.wait()
        plsc.subcore_barrier()

        def batch_quad(q, _):
            descs = []
            for u in range(4):
                b = q * 4 + u
                ib = ibufs[u]
                for g in range(8):
                    k16 = keyb[pl.ds(b * 128 + g * 16, 16)]
                    cb16 = auxb[pl.ds(b * 128 + g * 16, 16)]
                    local = k16 - cbase
                    m = (local >= 0) & (local < CBSZ)
                    dmp = CH_FLAT + t * 128 + g * 16 + lanes
                    ib[pl.ds(g * 16, 16)] = jnp.where(
                        m, local + cb16 * CBSZ, dmp)
                descs.append(pltpu.async_copy(
                    zb.at[pl.ds(b * 128, 128)], s_chunk.at[ib], dsem,
                    add=True))
            for dsc in descs:
                dsc.wait()
            return 0
        lax.fori_loop(0, NBATCH // 4, batch_quad, 0)
        for b in range(NBATCH - NBATCH % 4, NBATCH):
            for g in range(8):
                k16 = keyb[pl.ds(b * 128 + g * 16, 16)]
                cb16 = auxb[pl.ds(b * 128 + g * 16, 16)]
                local = k16 - cbase
                m = (local >= 0) & (local < CBSZ)
                dmp = CH_FLAT + t * 128 + g * 16 + lanes
                idxb0[pl.ds(g * 16, 16)] = jnp.where(m, local + cb16 * CBSZ,
                                                     dmp)
            pltpu.sync_copy(zb.at[pl.ds(b * 128, 128)],
                            s_chunk.at[idxb0], add=True)
        plsc.subcore_barrier()

        @pl.when(c == 0)
        def _():
            descs = [pltpu.async_copy(
                s_chunk.at[pl.ds(cb * CBSZ + t * CB_PT, CB_PT)],
                sa.at[pl.ds(ci * CBSZ + t * CB_PT, CB_PT)], dsem)
                for cb, sa in enumerate((s00, s01, s02, s03))]
            for dsc in descs:
                dsc.wait()

        @pl.when(c == 1)
        def _():
            descs = [pltpu.async_copy(
                s_chunk.at[pl.ds(cb * CBSZ + t * CB_PT, CB_PT)],
                sb_.at[pl.ds(ci * CBSZ + t * CB_PT, CB_PT)], dsem)
                for cb, sb_ in enumerate((s10, s11, s12, s13))]
            for dsc in descs:
                dsc.wait()
        plsc.subcore_barrier()
        return 0
    lax.fori_loop(0, NCHUNK, chunk_body, 0)


def _sc_scatter(edge_index, edge_type):
    mesh = plsc.VectorSubcoreMesh(core_axis_name="c", subcore_axis_name="s")
    kern = functools.partial(
        pl.kernel,
        mesh=mesh,
        out_type=[jax.ShapeDtypeStruct((SHALF,), jnp.float32)
                  for _ in range(8)],
        scratch_types=[
            pltpu.VMEM((EPT_PAD,), jnp.int32),
            pltpu.VMEM((EPT_PAD,), jnp.int32),
            pltpu.VMEM((EPT_PAD,), jnp.float32),
            pltpu.VMEM((ZU,), jnp.float32),
            pltpu.VMEM((128,), jnp.int32),
            pltpu.VMEM((128,), jnp.int32),
            pltpu.VMEM((128,), jnp.int32),
            pltpu.VMEM((128,), jnp.int32),
            pltpu.VMEM((128,), jnp.float32),
            pltpu.VMEM((640,), jnp.float32),
            pltpu.VMEM((256,), jnp.float32),
            pltpu.VMEM_SHARED((CH_TOT,), jnp.float32),
            pltpu.VMEM_SHARED((DEG_TOT,), jnp.float32),
            pltpu.SemaphoreType.DMA,
        ],
    )(_sc_scatter_kernel)
    zeros = jnp.zeros((CH_TOT,), jnp.float32)
    return kern(edge_index.reshape(-1), edge_type, zeros)


def _mm_bn_kernel(ri_ref, ro_ref, wi_ref, wo_ref,
                  si0, si1, si2, si3, so0, so1, so2, so3,
                  x_ref, stats_ref, ti_ref, to_ref, acc_ref):
    step = pl.program_id(0)

    @pl.when(step == 0)
    def _():
        ti_ref[...] = jnp.dot(ri_ref[...], wi_ref[...],
                              preferred_element_type=jnp.float32)
        to_ref[...] = jnp.dot(ro_ref[...], wo_ref[...],
                              preferred_element_type=jnp.float32)
        acc_ref[...] = jnp.zeros_like(acc_ref)

    xi = jnp.dot(si0[...], ti_ref[pl.ds(0, 128), :],
                 preferred_element_type=jnp.float32)
    xo = jnp.dot(so0[...], to_ref[pl.ds(0, 128), :],
                 preferred_element_type=jnp.float32)
    for cb, (si, so) in enumerate(((si1, so1), (si2, so2), (si3, so3)),
                                  start=1):
        xi = xi + jnp.dot(si[...], ti_ref[pl.ds(cb * 128, 128), :],
                          preferred_element_type=jnp.float32)
        xo = xo + jnp.dot(so[...], to_ref[pl.ds(cb * 128, 128), :],
                          preferred_element_type=jnp.float32)
    x = 0.5 * (xi + xo)
    x_ref[...] = x
    acc_ref[0, :] += jnp.sum(x, axis=0)
    acc_ref[1, :] += jnp.sum(x * x, axis=0)

    @pl.when(step == pl.num_programs(0) - 1)
    def _():
        stats_ref[...] = acc_ref[...]


def _bn_apply_kernel(x_ref, stats_ref, gamma_ref, beta_ref, out_ref):
    mean = stats_ref[0, :] * (1.0 / N_ENT)
    var = stats_ref[1, :] * (1.0 / N_ENT) - mean * mean
    scale = gamma_ref[...] * jax.lax.rsqrt(var + 1e-5)
    out_ref[...] = jnp.tanh((x_ref[...] - mean) * scale + beta_ref[...])


def _dense_stage(rel_in_p, rel_out_p, w_in, w_out, gamma, beta, s_parts):
    nb = N_ENT // ROW_BLK
    sblk = pl.BlockSpec((ROW_BLK, D), lambda i: (i, 0))
    x, stats = pl.pallas_call(
        _mm_bn_kernel,
        grid=(nb,),
        in_specs=[
            pl.BlockSpec((512, D), lambda i: (0, 0)),
            pl.BlockSpec((512, D), lambda i: (0, 0)),
            pl.BlockSpec((D, D), lambda i: (0, 0)),
            pl.BlockSpec((D, D), lambda i: (0, 0)),
        ] + [sblk] * 8,
        out_specs=[
            pl.BlockSpec((ROW_BLK, D), lambda i: (i, 0)),
            pl.BlockSpec((2, D), lambda i: (0, 0)),
        ],
        out_shape=[
            jax.ShapeDtypeStruct((N_ENT, D), jnp.float32),
            jax.ShapeDtypeStruct((2, D), jnp.float32),
        ],
        scratch_shapes=[
            pltpu.VMEM((512, D), jnp.float32),
            pltpu.VMEM((512, D), jnp.float32),
            pltpu.VMEM((2, D), jnp.float32),
        ],
    )(rel_in_p, rel_out_p, w_in, w_out, *s_parts)

    res = pl.pallas_call(
        _bn_apply_kernel,
        grid=(nb,),
        in_specs=[
            pl.BlockSpec((ROW_BLK, D), lambda i: (i, 0)),
            pl.BlockSpec((2, D), lambda i: (0, 0)),
            pl.BlockSpec((D,), lambda i: (0,)),
            pl.BlockSpec((D,), lambda i: (0,)),
        ],
        out_specs=pl.BlockSpec((ROW_BLK, D), lambda i: (i, 0)),
        out_shape=jax.ShapeDtypeStruct((N_ENT, D), jnp.float32),
    )(x, stats, gamma, beta)
    return res


def kernel(rel_embed, rel_embed_in, rel_embed_out, w_in, w_out, gamma, beta,
           edge_index, edge_type):
    outs = _sc_scatter(edge_index, edge_type)
    s_parts = [o.reshape(NPAD, D) for o in outs]
    pad = ((0, 512 - N_REL), (0, 0))
    rel_in_p = jnp.pad(rel_embed_in, pad)
    rel_out_p = jnp.pad(rel_embed_out, pad)
    res = _dense_stage(rel_in_p, rel_out_p, w_in, w_out, gamma, beta,
                       s_parts)
    return (res, rel_embed)


# speedup vs baseline: 1.4358x; 1.0382x over previous
import functools

import jax
import jax.numpy as jnp
from jax import lax
from jax.experimental import pallas as pl
from jax.experimental.pallas import tpu as pltpu
from jax.experimental.pallas import tpu_sc as plsc

N_ENT = 10000
N_REL = 500
D = 128
ROW_BLK = 2000
NPAD = 10240

NT = 16
E_HALF = 160000
EPT = E_HALF // NT
NBATCH = 79
EPT_PAD = NBATCH * 128

NCHUNK = 4
CH_N = NPAD // NCHUNK
CBSZ = CH_N * D
CH_FLAT = 4 * CBSZ
CH_TOT = CH_FLAT + NT * 128
CH_ZERO_PT = CH_FLAT // NT
ZU = CH_ZERO_PT // 16
CB_PT = CBSZ // NT
RD_UNIT = CB_PT // 2
SHALF = NPAD * D

DEG_TOT = 12288
DEG_PT = 10240
BIGKEY = 1 << 30


def _sc_scatter_kernel(ei_hbm, et_hbm, zc_hbm,
                       s00, s01, s02, s03, s10, s11, s12, s13,
                       keyb, auxb, zb, zerob, idxb0, idxb1, idxb2, idxb3,
                       idxb4, idxb5, idxb6, idxb7,
                       onesb, tmpb, ztb, s_chunk, deg_hist, dsem):
    c = lax.axis_index("c")
    t = lax.axis_index("s")
    ebase = c * E_HALF + t * EPT
    lanes = lax.iota(jnp.int32, 16)

    def zb_body(i, _):
        zerob[pl.ds(i * 16, 16)] = jnp.zeros((16,), jnp.float32)
        return 0
    lax.fori_loop(0, ZU // 16, zb_body, 0)
    for g in range(8):
        onesb[pl.ds(g * 16, 16)] = jnp.ones((16,), jnp.float32)

    pltpu.sync_copy(ei_hbm.at[pl.ds(ebase, EPT)], keyb.at[pl.ds(0, EPT)])
    for p in range(7):
        keyb[pl.ds(EPT + p * 16, 16)] = N_ENT + t * 128 + p * 16 + lanes

    pltpu.sync_copy(zerob.at[pl.ds(0, 768)],
                    deg_hist.at[pl.ds(t * 768, 768)])
    plsc.subcore_barrier()

    ibufs = (idxb0, idxb1, idxb2, idxb3, idxb4, idxb5, idxb6, idxb7)

    def hist_quad(q, _):
        descs = []
        for u in range(8):
            b = q * 8 + u
            ib = ibufs[u]
            for g in range(8):
                ib[pl.ds(g * 16, 16)] = keyb[pl.ds(b * 128 + g * 16, 16)]
            descs.append(pltpu.async_copy(onesb, deg_hist.at[ib], dsem,
                                          add=True))
        for dsc in descs:
            dsc.wait()
        return 0
    lax.fori_loop(0, NBATCH // 8, hist_quad, 0)
    for b in range(NBATCH - NBATCH % 8, NBATCH):
        for g in range(8):
            idxb0[pl.ds(g * 16, 16)] = keyb[pl.ds(b * 128 + g * 16, 16)]
        pltpu.sync_copy(onesb, deg_hist.at[idxb0], add=True)
    plsc.subcore_barrier()

    pltpu.sync_copy(deg_hist.at[pl.ds(t * 640, 640)], tmpb)

    def dinv_body(i, _):
        x = tmpb[pl.ds(i * 16, 16)]
        y = jnp.full((16,), 1.0, jnp.float32)
        for k in range(1, 11):
            y = jnp.where(x >= float(0.5 * 4 ** k), float(2.0 ** (-k)), y)
        for _ in range(5):
            y = y * (1.5 - 0.5 * x * y * y)
        tmpb[pl.ds(i * 16, 16)] = jnp.where(x >= 1.0, y, 0.0)
        return 0
    lax.fori_loop(0, 640 // 16, dinv_body, 0)
    pltpu.sync_copy(tmpb, deg_hist.at[pl.ds(t * 640, 640)])
    plsc.subcore_barrier()

    for p in range(7):
        auxb[pl.ds(EPT + p * 16, 16)] = jnp.zeros((16,), jnp.int32)
    pltpu.sync_copy(ei_hbm.at[pl.ds(2 * E_HALF + ebase, EPT)],
                    auxb.at[pl.ds(0, EPT)])

    def z_pair(q, _):
        descs = []
        for u in range(4):
            b = q * 4 + u
            ia, ik = ibufs[2 * u], ibufs[2 * u + 1]
            for g in range(8):
                ia[pl.ds(g * 16, 16)] = auxb[pl.ds(b * 128 + g * 16, 16)]
                ik[pl.ds(g * 16, 16)] = keyb[pl.ds(b * 128 + g * 16, 16)]
            descs.append(pltpu.async_copy(deg_hist.at[ia],
                                          zb.at[pl.ds(b * 128, 128)], dsem))
            descs.append(pltpu.async_copy(deg_hist.at[ik],
                                          ztb.at[pl.ds(u * 128, 128)], dsem))
        for dsc in descs:
            dsc.wait()
        for u in range(4):
            b = q * 4 + u
            for g in range(8):
                zb[pl.ds(b * 128 + g * 16, 16)] = (
                    zb[pl.ds(b * 128 + g * 16, 16)]
                    * ztb[pl.ds(u * 128 + g * 16, 16)])
        return 0
    lax.fori_loop(0, NBATCH // 4, z_pair, 0)
    for b in range(NBATCH - NBATCH % 4, NBATCH):
        for g in range(8):
            idxb0[pl.ds(g * 16, 16)] = auxb[pl.ds(b * 128 + g * 16, 16)]
            idxb1[pl.ds(g * 16, 16)] = keyb[pl.ds(b * 128 + g * 16, 16)]
        pltpu.sync_copy(deg_hist.at[idxb0], zb.at[pl.ds(b * 128, 128)])
        pltpu.sync_copy(deg_hist.at[idxb1], ztb.at[pl.ds(0, 128)])
        for g in range(8):
            zb[pl.ds(b * 128 + g * 16, 16)] = (
                zb[pl.ds(b * 128 + g * 16, 16)] * ztb[pl.ds(g * 16, 16)])

    pltpu.sync_copy(et_hbm.at[pl.ds(ebase, EPT)], auxb.at[pl.ds(0, EPT)])

    def key_body(i, _):
        ty = auxb[pl.ds(i * 16, 16)]
        keyb[pl.ds(i * 16, 16)] = (
            lax.shift_left(keyb[pl.ds(i * 16, 16)], 7) + (ty & 127))
        auxb[pl.ds(i * 16, 16)] = lax.shift_right_logical(ty, 7)
        return 0
    lax.fori_loop(0, EPT // 16, key_body, 0)
    for p in range(7):
        keyb[pl.ds(EPT + p * 16, 16)] = jnp.full((16,), BIGKEY, jnp.int32)
        auxb[pl.ds(EPT + p * 16, 16)] = jnp.zeros((16,), jnp.int32)

    def chunk_body(ci, _):
        cbase = ci * CBSZ

        zdescs = [pltpu.async_copy(
            zerob.at[pl.ds(0, ZU)],
            s_chunk.at[pl.ds(t * CH_ZERO_PT + j * ZU, ZU)], dsem)
            for j in range(CH_ZERO_PT // ZU)]
        for dsc in zdescs:
            dsc.wait()
        plsc.subcore_barrier()

        def batch_quad(q, _):
            descs = []
            for u in range(8):
                b = q * 8 + u
                ib = ibufs[u]
                for g in range(8):
                    k16 = keyb[pl.ds(b * 128 + g * 16, 16)]
                    cb16 = auxb[pl.ds(b * 128 + g * 16, 16)]
                    local = k16 - cbase
                    m = (local >= 0) & (local < CBSZ)
                    dmp = CH_FLAT + t * 128 + g * 16 + lanes
                    ib[pl.ds(g * 16, 16)] = jnp.where(
                        m, local + cb16 * CBSZ, dmp)
                descs.append(pltpu.async_copy(
                    zb.at[pl.ds(b * 128, 128)], s_chunk.at[ib], dsem,
                    add=True))
            for dsc in descs:
                dsc.wait()
            return 0
        lax.fori_loop(0, NBATCH // 8, batch_quad, 0)
        for b in range(NBATCH - NBATCH % 8, NBATCH):
            for g in range(8):
                k16 = keyb[pl.ds(b * 128 + g * 16, 16)]
                cb16 = auxb[pl.ds(b * 128 + g * 16, 16)]
                local = k16 - cbase
                m = (local >= 0) & (local < CBSZ)
                dmp = CH_FLAT + t * 128 + g * 16 + lanes
                idxb0[pl.ds(g * 16, 16)] = jnp.where(m, local + cb16 * CBSZ,
                                                     dmp)
            pltpu.sync_copy(zb.at[pl.ds(b * 128, 128)],
                            s_chunk.at[idxb0], add=True)
        plsc.subcore_barrier()

        @pl.when(c == 0)
        def _():
            descs = [pltpu.async_copy(
                s_chunk.at[pl.ds(cb * CBSZ + t * CB_PT, CB_PT)],
                sa.at[pl.ds(ci * CBSZ + t * CB_PT, CB_PT)], dsem)
                for cb, sa in enumerate((s00, s01, s02, s03))]
            for dsc in descs:
                dsc.wait()

        @pl.when(c == 1)
        def _():
            descs = [pltpu.async_copy(
                s_chunk.at[pl.ds(cb * CBSZ + t * CB_PT, CB_PT)],
                sb_.at[pl.ds(ci * CBSZ + t * CB_PT, CB_PT)], dsem)
                for cb, sb_ in enumerate((s10, s11, s12, s13))]
            for dsc in descs:
                dsc.wait()
        plsc.subcore_barrier()
        return 0
    lax.fori_loop(0, NCHUNK, chunk_body, 0)


def _sc_scatter(edge_index, edge_type):
    mesh = plsc.VectorSubcoreMesh(core_axis_name="c", subcore_axis_name="s")
    kern = functools.partial(
        pl.kernel,
        mesh=mesh,
        out_type=[jax.ShapeDtypeStruct((SHALF,), jnp.float32)
                  for _ in range(8)],
        scratch_types=[
            pltpu.VMEM((EPT_PAD,), jnp.int32),
            pltpu.VMEM((EPT_PAD,), jnp.int32),
            pltpu.VMEM((EPT_PAD,), jnp.float32),
            pltpu.VMEM((ZU,), jnp.float32),
            pltpu.VMEM((128,), jnp.int32),
            pltpu.VMEM((128,), jnp.int32),
            pltpu.VMEM((128,), jnp.int32),
            pltpu.VMEM((128,), jnp.int32),
            pltpu.VMEM((128,), jnp.int32),
            pltpu.VMEM((128,), jnp.int32),
            pltpu.VMEM((128,), jnp.int32),
            pltpu.VMEM((128,), jnp.int32),
            pltpu.VMEM((128,), jnp.float32),
            pltpu.VMEM((640,), jnp.float32),
            pltpu.VMEM((512,), jnp.float32),
            pltpu.VMEM_SHARED((CH_TOT,), jnp.float32),
            pltpu.VMEM_SHARED((DEG_TOT,), jnp.float32),
            pltpu.SemaphoreType.DMA,
        ],
    )(_sc_scatter_kernel)
    zeros = jnp.zeros((CH_TOT,), jnp.float32)
    return kern(edge_index.reshape(-1), edge_type, zeros)


def _mm_bn_kernel(ri_ref, ro_ref, wi_ref, wo_ref,
                  si0, si1, si2, si3, so0, so1, so2, so3,
                  x_ref, stats_ref, ti_ref, to_ref, acc_ref):
    step = pl.program_id(0)

    @pl.when(step == 0)
    def _():
        ti_ref[...] = jnp.dot(ri_ref[...], wi_ref[...],
                              preferred_element_type=jnp.float32)
        to_ref[...] = jnp.dot(ro_ref[...], wo_ref[...],
                              preferred_element_type=jnp.float32)
        acc_ref[...] = jnp.zeros_like(acc_ref)

    xi = jnp.dot(si0[...], ti_ref[pl.ds(0, 128), :],
                 preferred_element_type=jnp.float32)
    xo = jnp.dot(so0[...], to_ref[pl.ds(0, 128), :],
                 preferred_element_type=jnp.float32)
    for cb, (si, so) in enumerate(((si1, so1), (si2, so2), (si3, so3)),
                                  start=1):
        xi = xi + jnp.dot(si[...], ti_ref[pl.ds(cb * 128, 128), :],
                          preferred_element_type=jnp.float32)
        xo = xo + jnp.dot(so[...], to_ref[pl.ds(cb * 128, 128), :],
                          preferred_element_type=jnp.float32)
    x = 0.5 * (xi + xo)
    x_ref[...] = x
    acc_ref[0, :] += jnp.sum(x, axis=0)
    acc_ref[1, :] += jnp.sum(x * x, axis=0)

    @pl.when(step == pl.num_programs(0) - 1)
    def _():
        stats_ref[...] = acc_ref[...]


def _bn_apply_kernel(x_ref, stats_ref, gamma_ref, beta_ref, out_ref):
    mean = stats_ref[0, :] * (1.0 / N_ENT)
    var = stats_ref[1, :] * (1.0 / N_ENT) - mean * mean
    scale = gamma_ref[...] * jax.lax.rsqrt(var + 1e-5)
    out_ref[...] = jnp.tanh((x_ref[...] - mean) * scale + beta_ref[...])


def _dense_stage(rel_in_p, rel_out_p, w_in, w_out, gamma, beta, s_parts):
    nb = N_ENT // ROW_BLK
    sblk = pl.BlockSpec((ROW_BLK, D), lambda i: (i, 0))
    x, stats = pl.pallas_call(
        _mm_bn_kernel,
        grid=(nb,),
        in_specs=[
            pl.BlockSpec((512, D), lambda i: (0, 0)),
            pl.BlockSpec((512, D), lambda i: (0, 0)),
            pl.BlockSpec((D, D), lambda i: (0, 0)),
            pl.BlockSpec((D, D), lambda i: (0, 0)),
        ] + [sblk] * 8,
        out_specs=[
            pl.BlockSpec((ROW_BLK, D), lambda i: (i, 0)),
            pl.BlockSpec((2, D), lambda i: (0, 0)),
        ],
        out_shape=[
            jax.ShapeDtypeStruct((N_ENT, D), jnp.float32),
            jax.ShapeDtypeStruct((2, D), jnp.float32),
        ],
        scratch_shapes=[
            pltpu.VMEM((512, D), jnp.float32),
            pltpu.VMEM((512, D), jnp.float32),
            pltpu.VMEM((2, D), jnp.float32),
        ],
    )(rel_in_p, rel_out_p, w_in, w_out, *s_parts)

    res = pl.pallas_call(
        _bn_apply_kernel,
        grid=(nb,),
        in_specs=[
            pl.BlockSpec((ROW_BLK, D), lambda i: (i, 0)),
            pl.BlockSpec((2, D), lambda i: (0, 0)),
            pl.BlockSpec((D,), lambda i: (0,)),
            pl.BlockSpec((D,), lambda i: (0,)),
        ],
        out_specs=pl.BlockSpec((ROW_BLK, D), lambda i: (i, 0)),
        out_shape=jax.ShapeDtypeStruct((N_ENT, D), jnp.float32),
    )(x, stats, gamma, beta)
    return res


def kernel(rel_embed, rel_embed_in, rel_embed_out, w_in, w_out, gamma, beta,
           edge_index, edge_type):
    outs = _sc_scatter(edge_index, edge_type)
    s_parts = [o.reshape(NPAD, D) for o in outs]
    pad = ((0, 512 - N_REL), (0, 0))
    rel_in_p = jnp.pad(rel_embed_in, pad)
    rel_out_p = jnp.pad(rel_embed_out, pad)
    res = _dense_stage(rel_in_p, rel_out_p, w_in, w_out, gamma, beta,
                       s_parts)
    return (res, rel_embed)


# speedup vs baseline: 1.4634x; 1.0192x over previous
import functools

import jax
import jax.numpy as jnp
from jax import lax
from jax.experimental import pallas as pl
from jax.experimental.pallas import tpu as pltpu
from jax.experimental.pallas import tpu_sc as plsc

N_ENT = 10000
N_REL = 500
D = 128
ROW_BLK = 2000
NPAD = 10240

NT = 16
E_HALF = 160000
EPT = E_HALF // NT
NBATCH = 79
EPT_PAD = NBATCH * 128

NCHUNK = 4
CH_N = NPAD // NCHUNK
CBSZ = CH_N * D
CH_FLAT = 4 * CBSZ
CH_TOT = CH_FLAT + NT * 128
CH_ZERO_PT = CH_FLAT // NT
ZU = CH_ZERO_PT // 16
CB_PT = CBSZ // NT
RD_UNIT = CB_PT // 2
SHALF = NPAD * D

DEG_TOT = 12288
DEG_PT = 10240
BIGKEY = 1 << 30


def _sc_scatter_kernel(ei_hbm, et_hbm,
                       s00, s01, s02, s03, s10, s11, s12, s13,
                       keyb, auxb, zb, zerob, idxb0, idxb1, idxb2, idxb3,
                       idxb4, idxb5, idxb6, idxb7,
                       onesb, tmpb, ztb, s_chunk, deg_hist, dsem):
    c = lax.axis_index("c")
    t = lax.axis_index("s")
    ebase = c * E_HALF + t * EPT
    lanes = lax.iota(jnp.int32, 16)

    def zb_body(i, _):
        zerob[pl.ds(i * 16, 16)] = jnp.zeros((16,), jnp.float32)
        return 0
    lax.fori_loop(0, ZU // 16, zb_body, 0)
    for g in range(8):
        onesb[pl.ds(g * 16, 16)] = jnp.ones((16,), jnp.float32)

    pltpu.sync_copy(ei_hbm.at[pl.ds(ebase, EPT)], keyb.at[pl.ds(0, EPT)])
    for p in range(7):
        keyb[pl.ds(EPT + p * 16, 16)] = N_ENT + t * 128 + p * 16 + lanes

    pltpu.sync_copy(zerob.at[pl.ds(0, 768)],
                    deg_hist.at[pl.ds(t * 768, 768)])
    plsc.subcore_barrier()

    ibufs = (idxb0, idxb1, idxb2, idxb3, idxb4, idxb5, idxb6, idxb7)

    def hist_quad(q, _):
        descs = []
        for u in range(8):
            b = q * 8 + u
            ib = ibufs[u]
            for g in range(8):
                ib[pl.ds(g * 16, 16)] = keyb[pl.ds(b * 128 + g * 16, 16)]
            descs.append(pltpu.async_copy(onesb, deg_hist.at[ib], dsem,
                                          add=True))
        for dsc in descs:
            dsc.wait()
        return 0
    lax.fori_loop(0, NBATCH // 8, hist_quad, 0)
    for b in range(NBATCH - NBATCH % 8, NBATCH):
        for g in range(8):
            idxb0[pl.ds(g * 16, 16)] = keyb[pl.ds(b * 128 + g * 16, 16)]
        pltpu.sync_copy(onesb, deg_hist.at[idxb0], add=True)
    plsc.subcore_barrier()

    pltpu.sync_copy(deg_hist.at[pl.ds(t * 640, 640)], tmpb)

    def dinv_body(i, _):
        x = tmpb[pl.ds(i * 16, 16)]
        y = jnp.full((16,), 1.0, jnp.float32)
        for k in range(1, 11):
            y = jnp.where(x >= float(0.5 * 4 ** k), float(2.0 ** (-k)), y)
        for _ in range(5):
            y = y * (1.5 - 0.5 * x * y * y)
        tmpb[pl.ds(i * 16, 16)] = jnp.where(x >= 1.0, y, 0.0)
        return 0
    lax.fori_loop(0, 640 // 16, dinv_body, 0)
    pltpu.sync_copy(tmpb, deg_hist.at[pl.ds(t * 640, 640)])
    plsc.subcore_barrier()

    for p in range(7):
        auxb[pl.ds(EPT + p * 16, 16)] = jnp.zeros((16,), jnp.int32)
    pltpu.sync_copy(ei_hbm.at[pl.ds(2 * E_HALF + ebase, EPT)],
                    auxb.at[pl.ds(0, EPT)])

    def z_pair(q, _):
        descs = []
        for u in range(4):
            b = q * 4 + u
            ia, ik = ibufs[2 * u], ibufs[2 * u + 1]
            for g in range(8):
                ia[pl.ds(g * 16, 16)] = auxb[pl.ds(b * 128 + g * 16, 16)]
                ik[pl.ds(g * 16, 16)] = keyb[pl.ds(b * 128 + g * 16, 16)]
            descs.append(pltpu.async_copy(deg_hist.at[ia],
                                          zb.at[pl.ds(b * 128, 128)], dsem))
            descs.append(pltpu.async_copy(deg_hist.at[ik],
                                          ztb.at[pl.ds(u * 128, 128)], dsem))
        for dsc in descs:
            dsc.wait()
        for u in range(4):
            b = q * 4 + u
            for g in range(8):
                zb[pl.ds(b * 128 + g * 16, 16)] = (
                    zb[pl.ds(b * 128 + g * 16, 16)]
                    * ztb[pl.ds(u * 128 + g * 16, 16)])
        return 0
    lax.fori_loop(0, NBATCH // 4, z_pair, 0)
    for b in range(NBATCH - NBATCH % 4, NBATCH):
        for g in range(8):
            idxb0[pl.ds(g * 16, 16)] = auxb[pl.ds(b * 128 + g * 16, 16)]
            idxb1[pl.ds(g * 16, 16)] = keyb[pl.ds(b * 128 + g * 16, 16)]
        pltpu.sync_copy(deg_hist.at[idxb0], zb.at[pl.ds(b * 128, 128)])
        pltpu.sync_copy(deg_hist.at[idxb1], ztb.at[pl.ds(0, 128)])
        for g in range(8):
            zb[pl.ds(b * 128 + g * 16, 16)] = (
                zb[pl.ds(b * 128 + g * 16, 16)] * ztb[pl.ds(g * 16, 16)])

    pltpu.sync_copy(et_hbm.at[pl.ds(ebase, EPT)], auxb.at[pl.ds(0, EPT)])

    def key_body(i, _):
        ty = auxb[pl.ds(i * 16, 16)]
        keyb[pl.ds(i * 16, 16)] = (
            lax.shift_left(keyb[pl.ds(i * 16, 16)], 7) + (ty & 127))
        auxb[pl.ds(i * 16, 16)] = lax.shift_right_logical(ty, 7)
        return 0
    lax.fori_loop(0, EPT // 16, key_body, 0)
    for p in range(7):
        keyb[pl.ds(EPT + p * 16, 16)] = jnp.full((16,), BIGKEY, jnp.int32)
        auxb[pl.ds(EPT + p * 16, 16)] = jnp.zeros((16,), jnp.int32)

    def chunk_body(ci, _):
        cbase = ci * CBSZ

        zdescs = [pltpu.async_copy(
            zerob.at[pl.ds(0, ZU)],
            s_chunk.at[pl.ds(t * CH_ZERO_PT + j * ZU, ZU)], dsem)
            for j in range(CH_ZERO_PT // ZU)]
        for dsc in zdescs:
            dsc.wait()
        plsc.subcore_barrier()

        def batch_quad(q, _):
            descs = []
            for u in range(8):
                b = q * 8 + u
                ib = ibufs[u]
                for g in range(8):
                    k16 = keyb[pl.ds(b * 128 + g * 16, 16)]
                    cb16 = auxb[pl.ds(b * 128 + g * 16, 16)]
                    local = k16 - cbase
                    m = (local >= 0) & (local < CBSZ)
                    dmp = CH_FLAT + t * 128 + g * 16 + lanes
                    ib[pl.ds(g * 16, 16)] = jnp.where(
                        m, local + cb16 * CBSZ, dmp)
                descs.append(pltpu.async_copy(
                    zb.at[pl.ds(b * 128, 128)], s_chunk.at[ib], dsem,
                    add=True))
            for dsc in descs:
                dsc.wait()
            return 0
        lax.fori_loop(0, NBATCH // 8, batch_quad, 0)
        for b in range(NBATCH - NBATCH % 8, NBATCH):
            for g in range(8):
                k16 = keyb[pl.ds(b * 128 + g * 16, 16)]
                cb16 = auxb[pl.ds(b * 128 + g * 16, 16)]
                local = k16 - cbase
                m = (local >= 0) & (local < CBSZ)
                dmp = CH_FLAT + t * 128 + g * 16 + lanes
                idxb0[pl.ds(g * 16, 16)] = jnp.where(m, local + cb16 * CBSZ,
                                                     dmp)
            pltpu.sync_copy(zb.at[pl.ds(b * 128, 128)],
                            s_chunk.at[idxb0], add=True)
        plsc.subcore_barrier()

        @pl.when(c == 0)
        def _():
            descs = [pltpu.async_copy(
                s_chunk.at[pl.ds(cb * CBSZ + t * CB_PT, CB_PT)],
                sa.at[pl.ds(ci * CBSZ + t * CB_PT, CB_PT)], dsem)
                for cb, sa in enumerate((s00, s01, s02, s03))]
            for dsc in descs:
                dsc.wait()

        @pl.when(c == 1)
        def _():
            descs = [pltpu.async_copy(
                s_chunk.at[pl.ds(cb * CBSZ + t * CB_PT, CB_PT)],
                sb_.at[pl.ds(ci * CBSZ + t * CB_PT, CB_PT)], dsem)
                for cb, sb_ in enumerate((s10, s11, s12, s13))]
            for dsc in descs:
                dsc.wait()
        plsc.subcore_barrier()
        return 0
    lax.fori_loop(0, NCHUNK, chunk_body, 0)


def _sc_scatter(edge_index, edge_type):
    mesh = plsc.VectorSubcoreMesh(core_axis_name="c", subcore_axis_name="s")
    kern = functools.partial(
        pl.kernel,
        mesh=mesh,
        out_type=[jax.ShapeDtypeStruct((SHALF,), jnp.float32)
                  for _ in range(8)],
        scratch_types=[
            pltpu.VMEM((EPT_PAD,), jnp.int32),
            pltpu.VMEM((EPT_PAD,), jnp.int32),
            pltpu.VMEM((EPT_PAD,), jnp.float32),
            pltpu.VMEM((ZU,), jnp.float32),
            pltpu.VMEM((128,), jnp.int32),
            pltpu.VMEM((128,), jnp.int32),
            pltpu.VMEM((128,), jnp.int32),
            pltpu.VMEM((128,), jnp.int32),
            pltpu.VMEM((128,), jnp.int32),
            pltpu.VMEM((128,), jnp.int32),
            pltpu.VMEM((128,), jnp.int32),
            pltpu.VMEM((128,), jnp.int32),
            pltpu.VMEM((128,), jnp.float32),
            pltpu.VMEM((640,), jnp.float32),
            pltpu.VMEM((512,), jnp.float32),
            pltpu.VMEM_SHARED((CH_TOT,), jnp.float32),
            pltpu.VMEM_SHARED((DEG_TOT,), jnp.float32),
            pltpu.SemaphoreType.DMA,
        ],
    )(_sc_scatter_kernel)
    return kern(edge_index.reshape(-1), edge_type)


def _mm_bn_kernel(ri_ref, ro_ref, wi_ref, wo_ref,
                  si0, si1, si2, si3, so0, so1, so2, so3,
                  x_ref, stats_ref, ti_ref, to_ref, acc_ref):
    step = pl.program_id(0)

    @pl.when(step == 0)
    def _():
        ti_ref[...] = jnp.dot(ri_ref[...], wi_ref[...],
                              preferred_element_type=jnp.float32)
        to_ref[...] = jnp.dot(ro_ref[...], wo_ref[...],
                              preferred_element_type=jnp.float32)
        acc_ref[...] = jnp.zeros_like(acc_ref)

    xi = jnp.dot(si0[...], ti_ref[pl.ds(0, 128), :],
                 preferred_element_type=jnp.float32)
    xo = jnp.dot(so0[...], to_ref[pl.ds(0, 128), :],
                 preferred_element_type=jnp.float32)
    for cb, (si, so) in enumerate(((si1, so1), (si2, so2), (si3, so3)),
                                  start=1):
        xi = xi + jnp.dot(si[...], ti_ref[pl.ds(cb * 128, 128), :],
                          preferred_element_type=jnp.float32)
        xo = xo + jnp.dot(so[...], to_ref[pl.ds(cb * 128, 128), :],
                          preferred_element_type=jnp.float32)
    x = 0.5 * (xi + xo)
    x_ref[...] = x
    acc_ref[0, :] += jnp.sum(x, axis=0)
    acc_ref[1, :] += jnp.sum(x * x, axis=0)

    @pl.when(step == pl.num_programs(0) - 1)
    def _():
        stats_ref[...] = acc_ref[...]


def _bn_apply_kernel(x_ref, stats_ref, gamma_ref, beta_ref, out_ref):
    mean = stats_ref[0, :] * (1.0 / N_ENT)
    var = stats_ref[1, :] * (1.0 / N_ENT) - mean * mean
    scale = gamma_ref[...] * jax.lax.rsqrt(var + 1e-5)
    out_ref[...] = jnp.tanh((x_ref[...] - mean) * scale + beta_ref[...])


def _dense_stage(rel_in_p, rel_out_p, w_in, w_out, gamma, beta, s_parts):
    nb = N_ENT // ROW_BLK
    sblk = pl.BlockSpec((ROW_BLK, D), lambda i: (i, 0))
    x, stats = pl.pallas_call(
        _mm_bn_kernel,
        grid=(nb,),
        in_specs=[
            pl.BlockSpec((512, D), lambda i: (0, 0)),
            pl.BlockSpec((512, D), lambda i: (0, 0)),
            pl.BlockSpec((D, D), lambda i: (0, 0)),
            pl.BlockSpec((D, D), lambda i: (0, 0)),
        ] + [sblk] * 8,
        out_specs=[
            pl.BlockSpec((ROW_BLK, D), lambda i: (i, 0)),
            pl.BlockSpec((2, D), lambda i: (0, 0)),
        ],
        out_shape=[
            jax.ShapeDtypeStruct((N_ENT, D), jnp.float32),
            jax.ShapeDtypeStruct((2, D), jnp.float32),
        ],
        scratch_shapes=[
            pltpu.VMEM((512, D), jnp.float32),
            pltpu.VMEM((512, D), jnp.float32),
            pltpu.VMEM((2, D), jnp.float32),
        ],
    )(rel_in_p, rel_out_p, w_in, w_out, *s_parts)

    res = pl.pallas_call(
        _bn_apply_kernel,
        grid=(nb,),
        in_specs=[
            pl.BlockSpec((ROW_BLK, D), lambda i: (i, 0)),
            pl.BlockSpec((2, D), lambda i: (0, 0)),
            pl.BlockSpec((D,), lambda i: (0,)),
            pl.BlockSpec((D,), lambda i: (0,)),
        ],
        out_specs=pl.BlockSpec((ROW_BLK, D), lambda i: (i, 0)),
        out_shape=jax.ShapeDtypeStruct((N_ENT, D), jnp.float32),
    )(x, stats, gamma, beta)
    return res


def kernel(rel_embed, rel_embed_in, rel_embed_out, w_in, w_out, gamma, beta,
           edge_index, edge_type):
    outs = _sc_scatter(edge_index, edge_type)
    s_parts = [o.reshape(NPAD, D) for o in outs]
    pad = ((0, 512 - N_REL), (0, 0))
    rel_in_p = jnp.pad(rel_embed_in, pad)
    rel_out_p = jnp.pad(rel_embed_out, pad)
    res = _dense_stage(rel_in_p, rel_out_p, w_in, w_out, gamma, beta,
                       s_parts)
    return (res, rel_embed)
